# baseline (speedup 1.0000x reference)
import functools

import jax
import jax.numpy as jnp
from jax import lax
from jax.experimental import pallas as pl
from jax.experimental.pallas import tpu as pltpu
from jax.experimental.pallas import tpu_sc as plsc

N = 10000
D = 128
ALPHA = 0.2
K = 32
NC = 2
NS = 16
NW = NC * NS
C = 320
N_PAD = NW * C
GN = 4
NG = C // GN
NV = D // 16


NBUF = 4


def _sc_gather_mean_body(x_hbm, idx_hbm, out_hbm, idx_v, rows0, rows1, rows2,
                         rows3, out_v, sem0, sem1, sem2, sem3):
    c = lax.axis_index("c")
    s = lax.axis_index("s")
    rows = (rows0, rows1, rows2, rows3)
    sems = (sem0, sem1, sem2, sem3)
    wid = s * NC + c
    pltpu.sync_copy(idx_hbm.at[wid], idx_v)

    def compute(g, rbuf):
        for n in range(GN):
            def body(r, acc):
                return tuple(acc[j] + rbuf[n * K + r, pl.ds(j * 16, 16)]
                             for j in range(NV))
            acc0 = tuple(jnp.zeros((16,), jnp.float32) for _ in range(NV))
            acc = lax.fori_loop(0, K, body, acc0)
            for j in range(NV):
                out_v[g * GN + n, pl.ds(j * 16, 16)] = acc[j] * (1.0 / K)

    for b in range(NBUF):
        pltpu.async_copy(x_hbm.at[idx_v.at[b]], rows[b], sems[b])

    def step(q, carry):
        g0 = NBUF * q
        for b in range(NBUF):
            g = g0 + b
            pltpu.make_async_copy(x_hbm.at[idx_v.at[g]], rows[b],
                                  sems[b]).wait()
            compute(g, rows[b])

            @pl.when(g + NBUF < NG)
            def _():
                pltpu.async_copy(x_hbm.at[idx_v.at[g + NBUF]], rows[b],
                                 sems[b])
        return carry

    lax.fori_loop(0, NG // NBUF, step, 0)
    pltpu.sync_copy(out_v, out_hbm.at[wid])


@functools.partial(jax.jit)
def _sc_gather_mean(x, idx_grouped):
    mesh = plsc.VectorSubcoreMesh(core_axis_name="c", subcore_axis_name="s")
    out = pl.kernel(
        _sc_gather_mean_body,
        out_type=jax.ShapeDtypeStruct((NW, C, D), jnp.float32),
        mesh=mesh,
        scratch_types=[
            pltpu.VMEM((NG, 128), jnp.int32),
            pltpu.VMEM((GN * K, D), jnp.float32),
            pltpu.VMEM((GN * K, D), jnp.float32),
            pltpu.VMEM((GN * K, D), jnp.float32),
            pltpu.VMEM((GN * K, D), jnp.float32),
            pltpu.VMEM((C, D), jnp.float32),
            pltpu.SemaphoreType.DMA,
            pltpu.SemaphoreType.DMA,
            pltpu.SemaphoreType.DMA,
            pltpu.SemaphoreType.DMA,
        ],
    )(x, idx_grouped)
    return out.reshape(N_PAD, D)[:N]


def _dense_body(x_ref, agg_ref, h0_ref, h1_ref, wa_ref, wb_ref, whis_ref,
                wta_ref, wtb_ref, h_out, f_out):
    x = x_ref[...]
    agg = agg_ref[...]
    h_pre = jnp.dot(x, wa_ref[...], preferred_element_type=jnp.float32)
    h_pre = h_pre + jnp.dot(agg, wb_ref[...],
                            preferred_element_type=jnp.float32)
    h = jnp.where(h_pre >= 0, h_pre, ALPHA * h_pre)
    nrm = jnp.sqrt(jnp.sum(h * h, axis=1, keepdims=True))
    h = h / jnp.maximum(nrm, 1e-12)
    h_out[...] = h
    havg = (h0_ref[...] + h1_ref[...]) * 0.5
    tf = jnp.dot(havg, whis_ref[...], preferred_element_type=jnp.float32)
    f_pre = jnp.dot(h, wta_ref[...], preferred_element_type=jnp.float32)
    f_pre = f_pre + jnp.dot(tf, wtb_ref[...],
                            preferred_element_type=jnp.float32)
    f = jnp.where(f_pre >= 0, f_pre, ALPHA * f_pre)
    nrm2 = jnp.sqrt(jnp.sum(f * f, axis=1, keepdims=True))
    f_out[...] = f / jnp.maximum(nrm2, 1e-12)


BN = 1000


@jax.jit
def _dense(x, agg, h0, h1, wa, wb, whis, wta, wtb):
    row_spec = pl.BlockSpec((BN, D), lambda i: (i, 0))
    w_spec = pl.BlockSpec((D, D), lambda i: (0, 0))
    return pl.pallas_call(
        _dense_body,
        grid=(N // BN,),
        in_specs=[row_spec, row_spec, row_spec, row_spec,
                  w_spec, w_spec, w_spec, w_spec, w_spec],
        out_specs=[row_spec, row_spec],
        out_shape=[jax.ShapeDtypeStruct((N, D), jnp.float32),
                   jax.ShapeDtypeStruct((N, D), jnp.float32)],
    )(x, agg, h0, h1, wa, wb, whis, wta, wtb)


def _group_indices(idx):
    flat = idx.reshape(-1)
    flat = jnp.pad(flat, (0, N_PAD * K - flat.shape[0]))
    g = flat.reshape(NW, NG, 128)
    return jnp.tile(g[1:2], (NW, 1, 1))


def kernel(feats, agg_neigh_list1, agg_neigh_list2, history_hidden1,
           history_hidden2, W1, W2, W_his, W_T):
    idx1 = _group_indices(agg_neigh_list1)
    idx2 = _group_indices(agg_neigh_list2)
    wta, wtb = W_T[:D], W_T[D:]

    agg1 = _sc_gather_mean(feats, idx1)
    h1, f1 = _dense(feats, agg1, history_hidden1[0], history_hidden1[1],
                    W1[:D], W1[D:], W_his, wta, wtb)
    agg2 = _sc_gather_mean(f1, idx2)
    h2, feat = _dense(f1, agg2, history_hidden2[0], history_hidden2[1],
                      W2[:D], W2[D:], W_his, wta, wtb)
    return (h1, h2, feat)


# speedup vs baseline: 1.4983x; 1.4983x over previous
import functools

import jax
import jax.numpy as jnp
from jax import lax
from jax.experimental import pallas as pl
from jax.experimental.pallas import tpu as pltpu
from jax.experimental.pallas import tpu_sc as plsc

N = 10000
D = 128
ALPHA = 0.2
K = 32
NC = 2
NS = 16
NW = NC * NS
C = 320
N_PAD = NW * C
GN = 4
NG = C // GN
NV = D // 16


NBUF = 2
RPS = 624


NQ = NG // 2


def _sc_gather_mean_body(x_hbm, idx_hbm, out_hbm, spm, idx_v, rows0, rows1,
                         ob0, ob1, sem0, sem1, osem0, osem1):
    c = lax.axis_index("c")
    s = lax.axis_index("s")
    rows = (rows0, rows1)
    sems = (sem0, sem1)
    obufs = (ob0, ob1)
    osems = (osem0, osem1)
    wid = s * NC + c
    pltpu.sync_copy(x_hbm.at[pl.ds(s * RPS, RPS)], spm.at[pl.ds(s * RPS, RPS)])

    @pl.when(s == NS - 1)
    def _():
        pltpu.sync_copy(x_hbm.at[pl.ds(NS * RPS, N - NS * RPS)],
                        spm.at[pl.ds(NS * RPS, N - NS * RPS)])

    pltpu.sync_copy(idx_hbm.at[wid], idx_v)
    plsc.subcore_barrier()

    def compute(obuf, half, rbuf):
        for n in range(GN):
            def body(r, acc):
                return tuple(acc[j] + rbuf[n * K + r, pl.ds(j * 16, 16)]
                             for j in range(NV))
            acc0 = tuple(jnp.zeros((16,), jnp.float32) for _ in range(NV))
            acc = lax.fori_loop(0, K, body, acc0)
            for j in range(NV):
                obuf[half * GN + n, pl.ds(j * 16, 16)] = acc[j] * (1.0 / K)

    for b in range(NBUF):
        pltpu.async_copy(spm.at[idx_v.at[b]], rows[b], sems[b])

    def step(p, carry):
        for ob in range(2):
            q = 2 * p + ob

            @pl.when(q >= 2)
            def _():
                pltpu.make_async_copy(obufs[ob],
                                      out_hbm.at[wid * NQ + (q - 2)],
                                      osems[ob]).wait()

            for half in range(2):
                g = 2 * q + half
                b = half
                pltpu.make_async_copy(spm.at[idx_v.at[g]], rows[b],
                                      sems[b]).wait()
                compute(obufs[ob], half, rows[b])

                @pl.when(g + NBUF < NG)
                def _():
                    pltpu.async_copy(spm.at[idx_v.at[g + NBUF]], rows[b],
                                     sems[b])

            pltpu.async_copy(obufs[ob], out_hbm.at[wid * NQ + q], osems[ob])
        return carry

    lax.fori_loop(0, NQ // 2, step, 0)
    for q in (NQ - 2, NQ - 1):
        pltpu.make_async_copy(obufs[q % 2], out_hbm.at[wid * NQ + q],
                              osems[q % 2]).wait()


@functools.partial(jax.jit)
def _sc_gather_mean(x, idx_grouped):
    mesh = plsc.VectorSubcoreMesh(core_axis_name="c", subcore_axis_name="s")
    out = pl.kernel(
        _sc_gather_mean_body,
        out_type=jax.ShapeDtypeStruct((NW * NQ, 2 * GN, D), jnp.float32),
        mesh=mesh,
        scratch_types=[
            pltpu.VMEM_SHARED((N, D), jnp.float32),
            pltpu.VMEM((NG, 128), jnp.int32),
            pltpu.VMEM((GN * K, D), jnp.float32),
            pltpu.VMEM((GN * K, D), jnp.float32),
            pltpu.VMEM((2 * GN, D), jnp.float32),
            pltpu.VMEM((2 * GN, D), jnp.float32),
            pltpu.SemaphoreType.DMA,
            pltpu.SemaphoreType.DMA,
            pltpu.SemaphoreType.DMA,
            pltpu.SemaphoreType.DMA,
        ],
    )(x, idx_grouped)
    return out.reshape(N_PAD, D)[:N]


def _dense_body(x_ref, agg_ref, h0_ref, h1_ref, wa_ref, wb_ref, whis_ref,
                wta_ref, wtb_ref, h_out, f_out):
    x = x_ref[...]
    agg = agg_ref[...]
    h_pre = jnp.dot(x, wa_ref[...], preferred_element_type=jnp.float32)
    h_pre = h_pre + jnp.dot(agg, wb_ref[...],
                            preferred_element_type=jnp.float32)
    h = jnp.where(h_pre >= 0, h_pre, ALPHA * h_pre)
    nrm = jnp.sqrt(jnp.sum(h * h, axis=1, keepdims=True))
    h = h / jnp.maximum(nrm, 1e-12)
    h_out[...] = h
    havg = (h0_ref[...] + h1_ref[...]) * 0.5
    tf = jnp.dot(havg, whis_ref[...], preferred_element_type=jnp.float32)
    f_pre = jnp.dot(h, wta_ref[...], preferred_element_type=jnp.float32)
    f_pre = f_pre + jnp.dot(tf, wtb_ref[...],
                            preferred_element_type=jnp.float32)
    f = jnp.where(f_pre >= 0, f_pre, ALPHA * f_pre)
    nrm2 = jnp.sqrt(jnp.sum(f * f, axis=1, keepdims=True))
    f_out[...] = f / jnp.maximum(nrm2, 1e-12)


BN = 1000


@jax.jit
def _dense(x, agg, h0, h1, wa, wb, whis, wta, wtb):
    row_spec = pl.BlockSpec((BN, D), lambda i: (i, 0))
    w_spec = pl.BlockSpec((D, D), lambda i: (0, 0))
    return pl.pallas_call(
        _dense_body,
        grid=(N // BN,),
        in_specs=[row_spec, row_spec, row_spec, row_spec,
                  w_spec, w_spec, w_spec, w_spec, w_spec],
        out_specs=[row_spec, row_spec],
        out_shape=[jax.ShapeDtypeStruct((N, D), jnp.float32),
                   jax.ShapeDtypeStruct((N, D), jnp.float32)],
    )(x, agg, h0, h1, wa, wb, whis, wta, wtb)


def _group_indices(idx):
    flat = idx.reshape(-1)
    flat = jnp.pad(flat, (0, N_PAD * K - flat.shape[0]))
    return flat.reshape(NW, NG, 128)


def kernel(feats, agg_neigh_list1, agg_neigh_list2, history_hidden1,
           history_hidden2, W1, W2, W_his, W_T):
    idx1 = _group_indices(agg_neigh_list1)
    idx2 = _group_indices(agg_neigh_list2)
    wta, wtb = W_T[:D], W_T[D:]

    agg1 = _sc_gather_mean(feats, idx1)
    h1, f1 = _dense(feats, agg1, history_hidden1[0], history_hidden1[1],
                    W1[:D], W1[D:], W_his, wta, wtb)
    agg2 = _sc_gather_mean(f1, idx2)
    h2, feat = _dense(f1, agg2, history_hidden2[0], history_hidden2[1],
                      W2[:D], W2[D:], W_his, wta, wtb)
    return (h1, h2, feat)


# speedup vs baseline: 1.6130x; 1.0765x over previous
import functools

import jax
import jax.numpy as jnp
from jax import lax
from jax.experimental import pallas as pl
from jax.experimental.pallas import tpu as pltpu
from jax.experimental.pallas import tpu_sc as plsc

N = 10000
D = 128
ALPHA = 0.2
K = 32
NC = 2
NS = 16
NW = NC * NS
C = 320
N_PAD = NW * C
GN = 4
NG = C // GN
NV = D // 16


NBUF = 2
RPS = 624


NQ = NG // 2


def _sc_gather_mean_body(x_hbm, idx_hbm, out_hbm, spm, idx_v, rows0, rows1,
                         ob0, ob1, sem0, sem1, osem0, osem1):
    c = lax.axis_index("c")
    s = lax.axis_index("s")
    rows = (rows0, rows1)
    sems = (sem0, sem1)
    obufs = (ob0, ob1)
    osems = (osem0, osem1)
    wid = s * NC + c
    pltpu.sync_copy(x_hbm.at[pl.ds(s * RPS, RPS)], spm.at[pl.ds(s * RPS, RPS)])

    @pl.when(s == NS - 1)
    def _():
        pltpu.sync_copy(x_hbm.at[pl.ds(NS * RPS, N - NS * RPS)],
                        spm.at[pl.ds(NS * RPS, N - NS * RPS)])

    pltpu.sync_copy(idx_hbm.at[wid], idx_v)
    plsc.subcore_barrier()

    def compute(obuf, half, rbuf):
        for n in range(GN):
            def body(r, acc):
                return tuple(acc[j] + rbuf[n * K + r, pl.ds(j * 16, 16)]
                             for j in range(NV))
            acc0 = tuple(jnp.zeros((16,), jnp.float32) for _ in range(NV))
            acc = lax.fori_loop(0, K, body, acc0)
            for j in range(NV):
                obuf[half * GN + n, pl.ds(j * 16, 16)] = acc[j] * (1.0 / K)

    for b in range(NBUF):
        pltpu.async_copy(spm.at[idx_v.at[b]], rows[b], sems[b])

    def step(p, carry):
        for ob in range(2):
            q = 2 * p + ob

            @pl.when(q >= 2)
            def _():
                pltpu.make_async_copy(obufs[ob],
                                      out_hbm.at[wid * NQ + (q - 2)],
                                      osems[ob]).wait()

            for half in range(2):
                g = 2 * q + half
                b = half
                pltpu.make_async_copy(spm.at[idx_v.at[g]], rows[b],
                                      sems[b]).wait()
                compute(obufs[ob], half, rows[b])

                @pl.when(g + NBUF < NG)
                def _():
                    pltpu.async_copy(spm.at[idx_v.at[g + NBUF]], rows[b],
                                     sems[b])

            pltpu.async_copy(obufs[ob], out_hbm.at[wid * NQ + q], osems[ob])
        return carry

    lax.fori_loop(0, NQ // 2, step, 0)
    for q in (NQ - 2, NQ - 1):
        pltpu.make_async_copy(obufs[q % 2], out_hbm.at[wid * NQ + q],
                              osems[q % 2]).wait()


@functools.partial(jax.jit)
def _sc_gather_mean(x, idx_grouped):
    mesh = plsc.VectorSubcoreMesh(core_axis_name="c", subcore_axis_name="s")
    out = pl.kernel(
        _sc_gather_mean_body,
        out_type=jax.ShapeDtypeStruct((NW * NQ, 2 * GN, D), jnp.float32),
        mesh=mesh,
        scratch_types=[
            pltpu.VMEM_SHARED((N, D), jnp.float32),
            pltpu.VMEM((NG, 128), jnp.int32),
            pltpu.VMEM((GN * K, D), jnp.float32),
            pltpu.VMEM((GN * K, D), jnp.float32),
            pltpu.VMEM((2 * GN, D), jnp.float32),
            pltpu.VMEM((2 * GN, D), jnp.float32),
            pltpu.SemaphoreType.DMA,
            pltpu.SemaphoreType.DMA,
            pltpu.SemaphoreType.DMA,
            pltpu.SemaphoreType.DMA,
        ],
    )(x, idx_grouped)
    return out.reshape(N_PAD, D)


def _tf_body(h0_ref, h1_ref, whis_ref, tf_out):
    havg = (h0_ref[...] + h1_ref[...]) * 0.5
    tf_out[...] = jnp.dot(havg, whis_ref[...],
                          preferred_element_type=jnp.float32)


def _dense_body(x_ref, agg_ref, tf_ref, wa_ref, wb_ref,
                wta_ref, wtb_ref, h_out, f_out):
    x = x_ref[...]
    agg = agg_ref[...]
    h_pre = jnp.dot(x, wa_ref[...], preferred_element_type=jnp.float32)
    h_pre = h_pre + jnp.dot(agg, wb_ref[...],
                            preferred_element_type=jnp.float32)
    h = jnp.where(h_pre >= 0, h_pre, ALPHA * h_pre)
    nrm = jnp.sqrt(jnp.sum(h * h, axis=1, keepdims=True))
    h = h / jnp.maximum(nrm, 1e-12)
    h_out[...] = h
    f_pre = jnp.dot(h, wta_ref[...], preferred_element_type=jnp.float32)
    f_pre = f_pre + jnp.dot(tf_ref[...], wtb_ref[...],
                            preferred_element_type=jnp.float32)
    f = jnp.where(f_pre >= 0, f_pre, ALPHA * f_pre)
    nrm2 = jnp.sqrt(jnp.sum(f * f, axis=1, keepdims=True))
    f_out[...] = f / jnp.maximum(nrm2, 1e-12)


BN = 1000
_ROW_SPEC = pl.BlockSpec((BN, D), lambda i: (i, 0))
_W_SPEC = pl.BlockSpec((D, D), lambda i: (0, 0))


@jax.jit
def _tf(h0, h1, whis):
    return pl.pallas_call(
        _tf_body,
        grid=(N // BN,),
        in_specs=[_ROW_SPEC, _ROW_SPEC, _W_SPEC],
        out_specs=_ROW_SPEC,
        out_shape=jax.ShapeDtypeStruct((N, D), jnp.float32),
    )(h0, h1, whis)


@jax.jit
def _dense(x, agg, tf, wa, wb, wta, wtb):
    return pl.pallas_call(
        _dense_body,
        grid=(N // BN,),
        in_specs=[_ROW_SPEC, _ROW_SPEC, _ROW_SPEC,
                  _W_SPEC, _W_SPEC, _W_SPEC, _W_SPEC],
        out_specs=[_ROW_SPEC, _ROW_SPEC],
        out_shape=[jax.ShapeDtypeStruct((N, D), jnp.float32),
                   jax.ShapeDtypeStruct((N, D), jnp.float32)],
    )(x, agg, tf, wa, wb, wta, wtb)


def _group_indices(idx):
    flat = idx.reshape(-1)
    flat = jnp.pad(flat, (0, N_PAD * K - flat.shape[0]))
    return flat.reshape(NW, NG, 128)


def kernel(feats, agg_neigh_list1, agg_neigh_list2, history_hidden1,
           history_hidden2, W1, W2, W_his, W_T):
    idx1 = _group_indices(agg_neigh_list1)
    idx2 = _group_indices(agg_neigh_list2)
    wta, wtb = W_T[:D], W_T[D:]

    agg1 = _sc_gather_mean(feats, idx1)
    tf1 = _tf(history_hidden1[0], history_hidden1[1], W_his)
    tf2 = _tf(history_hidden2[0], history_hidden2[1], W_his)
    h1, f1 = _dense(feats, agg1, tf1, W1[:D], W1[D:], wta, wtb)
    agg2 = _sc_gather_mean(f1, idx2)
    h2, feat = _dense(f1, agg2, tf2, W2[:D], W2[D:], wta, wtb)
    return (h1, h2, feat)


# speedup vs baseline: 1.6359x; 1.0142x over previous
import functools

import jax
import jax.numpy as jnp
from jax import lax
from jax.experimental import pallas as pl
from jax.experimental.pallas import tpu as pltpu
from jax.experimental.pallas import tpu_sc as plsc

N = 10000
D = 128
ALPHA = 0.2
K = 32
NC = 2
NS = 16
NW = NC * NS
C = 320
N_PAD = NW * C
GN = 4
NG = C // GN
NV = D // 16


NBUF = 2
RPS = 624


NQ = NG // 2


def _sc_gather_mean_body(x_hbm, idx_hbm, out_hbm, spm, idx_v, rows0, rows1,
                         ob0, ob1, sem0, sem1, osem0, osem1):
    c = lax.axis_index("c")
    s = lax.axis_index("s")
    rows = (rows0, rows1)
    sems = (sem0, sem1)
    obufs = (ob0, ob1)
    osems = (osem0, osem1)
    wid = s * NC + c
    npairs = jnp.where(wid == NW - 1, (N - (NW - 1) * C) // 8, NQ)
    ngroups = 2 * npairs
    pltpu.sync_copy(x_hbm.at[pl.ds(s * RPS, RPS)], spm.at[pl.ds(s * RPS, RPS)])

    @pl.when(s == NS - 1)
    def _():
        pltpu.sync_copy(x_hbm.at[pl.ds(NS * RPS, N - NS * RPS)],
                        spm.at[pl.ds(NS * RPS, N - NS * RPS)])

    @pl.when(wid < NW - 1)
    def _():
        pltpu.sync_copy(idx_hbm.at[pl.ds(wid * (C * K), C * K)], idx_v)

    @pl.when(wid == NW - 1)
    def _():
        rem = (N - (NW - 1) * C) * K
        pltpu.sync_copy(idx_hbm.at[pl.ds((NW - 1) * (C * K), rem)],
                        idx_v.at[pl.ds(0, rem)])

    plsc.subcore_barrier()

    def compute(obuf, half, rbuf):
        for n in range(GN):
            def body(r, acc):
                return tuple(acc[j] + rbuf[n * K + r, pl.ds(j * 16, 16)]
                             for j in range(NV))
            acc0 = tuple(jnp.zeros((16,), jnp.float32) for _ in range(NV))
            acc = lax.fori_loop(0, K, body, acc0)
            for j in range(NV):
                obuf[half * GN + n, pl.ds(j * 16, 16)] = acc[j] * (1.0 / K)

    def gidx(g):
        return idx_v.at[pl.ds(g * 128, 128)]

    for b in range(NBUF):
        pltpu.async_copy(spm.at[gidx(b)], rows[b], sems[b])

    def step(p, carry):
        for ob in range(2):
            q = 2 * p + ob

            @pl.when(q >= 2)
            def _():
                pltpu.make_async_copy(obufs[ob],
                                      out_hbm.at[wid * NQ + (q - 2)],
                                      osems[ob]).wait()

            for half in range(2):
                g = 2 * q + half
                b = half
                pltpu.make_async_copy(spm.at[gidx(g)], rows[b],
                                      sems[b]).wait()
                compute(obufs[ob], half, rows[b])

                @pl.when(g + NBUF < ngroups)
                def _():
                    pltpu.async_copy(spm.at[gidx(g + NBUF)], rows[b],
                                     sems[b])

            pltpu.async_copy(obufs[ob], out_hbm.at[wid * NQ + q], osems[ob])
        return carry

    lax.fori_loop(0, npairs // 2, step, 0)
    pltpu.make_async_copy(obufs[0], out_hbm.at[wid * NQ + npairs - 2],
                          osems[0]).wait()
    pltpu.make_async_copy(obufs[1], out_hbm.at[wid * NQ + npairs - 1],
                          osems[1]).wait()


@functools.partial(jax.jit)
def _sc_gather_mean(x, idx_grouped):
    mesh = plsc.VectorSubcoreMesh(core_axis_name="c", subcore_axis_name="s")
    out = pl.kernel(
        _sc_gather_mean_body,
        out_type=jax.ShapeDtypeStruct((NW * NQ, 2 * GN, D), jnp.float32),
        mesh=mesh,
        scratch_types=[
            pltpu.VMEM_SHARED((N, D), jnp.float32),
            pltpu.VMEM((C * K,), jnp.int32),
            pltpu.VMEM((GN * K, D), jnp.float32),
            pltpu.VMEM((GN * K, D), jnp.float32),
            pltpu.VMEM((2 * GN, D), jnp.float32),
            pltpu.VMEM((2 * GN, D), jnp.float32),
            pltpu.SemaphoreType.DMA,
            pltpu.SemaphoreType.DMA,
            pltpu.SemaphoreType.DMA,
            pltpu.SemaphoreType.DMA,
        ],
    )(x, idx_grouped)
    return out.reshape(N_PAD, D)


def _tf_body(h0_ref, h1_ref, whis_ref, tf_out):
    havg = (h0_ref[...] + h1_ref[...]) * 0.5
    tf_out[...] = jnp.dot(havg, whis_ref[...],
                          preferred_element_type=jnp.float32)


def _dense_body(x_ref, agg_ref, tf_ref, wa_ref, wb_ref,
                wta_ref, wtb_ref, h_out, f_out):
    x = x_ref[...]
    agg = agg_ref[...]
    h_pre = jnp.dot(x, wa_ref[...], preferred_element_type=jnp.float32)
    h_pre = h_pre + jnp.dot(agg, wb_ref[...],
                            preferred_element_type=jnp.float32)
    h = jnp.where(h_pre >= 0, h_pre, ALPHA * h_pre)
    nrm = jnp.sqrt(jnp.sum(h * h, axis=1, keepdims=True))
    h = h / jnp.maximum(nrm, 1e-12)
    h_out[...] = h
    f_pre = jnp.dot(h, wta_ref[...], preferred_element_type=jnp.float32)
    f_pre = f_pre + jnp.dot(tf_ref[...], wtb_ref[...],
                            preferred_element_type=jnp.float32)
    f = jnp.where(f_pre >= 0, f_pre, ALPHA * f_pre)
    nrm2 = jnp.sqrt(jnp.sum(f * f, axis=1, keepdims=True))
    f_out[...] = f / jnp.maximum(nrm2, 1e-12)


BN = 1000
_ROW_SPEC = pl.BlockSpec((BN, D), lambda i: (i, 0))
_W_SPEC = pl.BlockSpec((D, D), lambda i: (0, 0))


@jax.jit
def _tf(h0, h1, whis):
    return pl.pallas_call(
        _tf_body,
        grid=(N // BN,),
        in_specs=[_ROW_SPEC, _ROW_SPEC, _W_SPEC],
        out_specs=_ROW_SPEC,
        out_shape=jax.ShapeDtypeStruct((N, D), jnp.float32),
    )(h0, h1, whis)


@jax.jit
def _dense(x, agg, tf, wa, wb, wta, wtb):
    return pl.pallas_call(
        _dense_body,
        grid=(N // BN,),
        in_specs=[_ROW_SPEC, _ROW_SPEC, _ROW_SPEC,
                  _W_SPEC, _W_SPEC, _W_SPEC, _W_SPEC],
        out_specs=[_ROW_SPEC, _ROW_SPEC],
        out_shape=[jax.ShapeDtypeStruct((N, D), jnp.float32),
                   jax.ShapeDtypeStruct((N, D), jnp.float32)],
    )(x, agg, tf, wa, wb, wta, wtb)


def kernel(feats, agg_neigh_list1, agg_neigh_list2, history_hidden1,
           history_hidden2, W1, W2, W_his, W_T):
    idx1 = agg_neigh_list1.reshape(-1)
    idx2 = agg_neigh_list2.reshape(-1)
    wta, wtb = W_T[:D], W_T[D:]

    agg1 = _sc_gather_mean(feats, idx1)
    tf1 = _tf(history_hidden1[0], history_hidden1[1], W_his)
    tf2 = _tf(history_hidden2[0], history_hidden2[1], W_his)
    h1, f1 = _dense(feats, agg1, tf1, W1[:D], W1[D:], wta, wtb)
    agg2 = _sc_gather_mean(f1, idx2)
    h2, feat = _dense(f1, agg2, tf2, W2[:D], W2[D:], wta, wtb)
    return (h1, h2, feat)
